# baseline (speedup 1.0000x reference)
import functools

import jax
import jax.numpy as jnp
from jax import lax
from jax.experimental import pallas as pl
from jax.experimental.pallas import tpu as pltpu
from jax.experimental.pallas import tpu_sc as plsc

N = 10000
NP = 10240
E = 160000
D_IN = 256
DH = 512
DHL = 1024
DOUT = 128
G = 64

NC, NS, L = 2, 16, 16
NW = NC * NS
HDH = DH // 2
FPP = 4
NPASS = 2
ROWW = FPP * NP
PKROW = HDH // FPP
NROW = DH // FPP
EC = E // NW
CHUNK = 3200
NCHUNK = E // CHUNK
SHIFT = 14
BN = 1024

_sc_params = pltpu.CompilerParams(needs_layout_passes=False)


def _vmesh():
    return plsc.VectorSubcoreMesh(core_axis_name="c", subcore_axis_name="s",
                                  num_cores=NC, num_subcores=NS)


@functools.cache
def _sc_degree_fn():
    return functools.partial(
        pl.kernel,
        out_type=jax.ShapeDtypeStruct((NW, NP), jnp.float32),
        mesh=_vmesh(),
        scratch_types=[pltpu.VMEM((NP,), jnp.float32),
                       pltpu.VMEM((EC + L,), jnp.int32)],
        compiler_params=_sc_params,
    )(_sc_degree_body)


def _sc_degree_body(dst_hbm, out_hbm, cnt_v, dst_v):
    wid = lax.axis_index("s") * NC + lax.axis_index("c")

    def zb(i, c):
        cnt_v[pl.ds(i * L, L)] = jnp.zeros((L,), jnp.float32)
        return c
    lax.fori_loop(0, NP // L, zb, 0)

    tail = (EC // L) * L
    dst_v[pl.ds(tail, L)] = jnp.zeros((L,), jnp.int32)
    pltpu.sync_copy(dst_hbm.at[pl.ds(wid * EC, EC)], dst_v.at[pl.ds(0, EC)])

    ones = jnp.ones((L,), jnp.float32)

    def body(g, c):
        d = dst_v[pl.ds(g * L, L)]
        plsc.addupdate_scatter(cnt_v, [d], ones)
        return c
    lax.fori_loop(0, EC // L, body, 0)
    dtl = dst_v[pl.ds(tail, L)]
    mask = lax.iota(jnp.int32, L) < (EC % L)
    plsc.addupdate_scatter(cnt_v, [dtl], ones, mask=mask)

    pltpu.sync_copy(cnt_v, out_hbm.at[wid])


@functools.cache
def _sc_agg_fn():
    return functools.partial(
        pl.kernel,
        out_type=jax.ShapeDtypeStruct((NROW, ROWW), jnp.float32),
        mesh=_vmesh(),
        scratch_types=[pltpu.VMEM((ROWW,), jnp.int32),
                       pltpu.VMEM((2 * ROWW,), jnp.float32),
                       pltpu.VMEM((CHUNK,), jnp.int32),
                       pltpu.VMEM((CHUNK,), jnp.int32),
                       pltpu.SemaphoreType.DMA,
                       pltpu.SemaphoreType.DMA],
        compiler_params=_sc_params,
    )(_sc_agg_body)


def _sc_agg_body(y_hbm, pk_hbm, out_hbm, y_v, acc_v, eb0, eb1, sem0, sem1):
    wid = lax.axis_index("s") * NC + lax.axis_index("c")

    def process(buf):
        @plsc.parallel_loop(0, CHUNK // L, unroll=4)
        def _(g2):
            e = buf[pl.ds(g2 * L, L)]
            s = e & (2 ** SHIFT - 1)
            d = e >> SHIFT
            for pr in range(FPP):
                g = plsc.load_gather(y_v, [s + (pr * NP)])
                flo = plsc.bitcast(g << 16, jnp.float32)
                fhi = plsc.bitcast(g & jnp.int32(-65536), jnp.float32)
                plsc.addupdate_scatter(acc_v, [d + (pr * NP)], flo)
                plsc.addupdate_scatter(acc_v, [d + ((FPP + pr) * NP)], fhi)

    def pass_body(p, c0):
        row = wid * NPASS + p
        pltpu.make_async_copy(pk_hbm.at[pl.ds(0, CHUNK)], eb0, sem0).start()
        pltpu.sync_copy(y_hbm.at[row], y_v)

        @plsc.parallel_loop(0, 2 * ROWW // L, unroll=8)
        def _(i):
            acc_v[pl.ds(i * L, L)] = jnp.zeros((L,), jnp.float32)

        def chunk_pair(i, c):
            ch0 = i * 2
            pltpu.make_async_copy(
                pk_hbm.at[pl.ds((ch0 + 1) * CHUNK, CHUNK)], eb1, sem1).start()
            pltpu.make_async_copy(
                pk_hbm.at[pl.ds(ch0 * CHUNK, CHUNK)], eb0, sem0).wait()
            process(eb0)
            pltpu.make_async_copy(
                pk_hbm.at[pl.ds((ch0 + 2) * CHUNK, CHUNK)], eb0, sem0).start()
            pltpu.make_async_copy(
                pk_hbm.at[pl.ds((ch0 + 1) * CHUNK, CHUNK)], eb1, sem1).wait()
            process(eb1)
            return c
        lax.fori_loop(0, NCHUNK // 2, chunk_pair, 0)
        pltpu.make_async_copy(
            pk_hbm.at[pl.ds(NCHUNK * CHUNK, CHUNK)], eb0, sem0).wait()

        pltpu.sync_copy(acc_v.at[pl.ds(0, ROWW)], out_hbm.at[row])
        pltpu.sync_copy(acc_v.at[pl.ds(ROWW, ROWW)], out_hbm.at[PKROW + row])
        return c0
    lax.fori_loop(0, NPASS, pass_body, 0)


def _pack_pairs(y):
    v = lax.bitcast_convert_type(y, jnp.uint32)
    r = (v + jnp.uint32(0x7FFF) + ((v >> 16) & jnp.uint32(1))) >> 16
    pk = r[:HDH, :] | (r[HDH:, :] << 16)
    return lax.bitcast_convert_type(pk, jnp.int32)


def _unpack_pairs(pk):
    u = lax.bitcast_convert_type(pk, jnp.uint32)
    lo = lax.bitcast_convert_type(u << 16, jnp.float32)
    hi = lax.bitcast_convert_type(u & jnp.uint32(0xFFFF0000), jnp.float32)
    return jnp.concatenate([lo, hi], axis=0)


def _tc_first_body(x_ref, w_ref, cnt_ref, y_ref, dis_ref):
    c = jnp.sum(cnt_ref[...], axis=0, keepdims=True)
    d = lax.rsqrt(c + 1.0)
    y = lax.dot_general(w_ref[...], x_ref[...], (((0,), (1,)), ((), ())),
                        preferred_element_type=jnp.float32)
    y_ref[...] = _pack_pairs(y * d)
    dis_ref[...] = d


def _tc_first(x_p, W1, cnt_parts):
    return pl.pallas_call(
        _tc_first_body,
        grid=(NP // BN,),
        in_specs=[pl.BlockSpec((BN, D_IN), lambda i: (i, 0)),
                  pl.BlockSpec((D_IN, DH), lambda i: (0, 0)),
                  pl.BlockSpec((NW, BN), lambda i: (0, i))],
        out_specs=[pl.BlockSpec((HDH, BN), lambda i: (0, i)),
                   pl.BlockSpec((1, BN), lambda i: (0, i))],
        out_shape=[jax.ShapeDtypeStruct((HDH, NP), jnp.int32),
                   jax.ShapeDtypeStruct((1, NP), jnp.float32)],
    )(x_p, W1, cnt_parts)


def _tc_mid_body(agg_ref, y_ref, dis_ref, b_ref, w_ref, out_ref):
    d = dis_ref[...]
    y = _unpack_pairs(y_ref[...])
    h = jax.nn.gelu((agg_ref[...] + y) * d + b_ref[...])
    yn = lax.dot_general(w_ref[...], h, (((0,), (0,)), ((), ())),
                         preferred_element_type=jnp.float32)
    out_ref[...] = _pack_pairs(yn * d)


def _tc_mid(aggT, ypk, dis, b_col, Wn):
    return pl.pallas_call(
        _tc_mid_body,
        grid=(NP // BN,),
        in_specs=[pl.BlockSpec((DH, BN), lambda i: (0, i)),
                  pl.BlockSpec((HDH, BN), lambda i: (0, i)),
                  pl.BlockSpec((1, BN), lambda i: (0, i)),
                  pl.BlockSpec((DH, 1), lambda i: (0, 0)),
                  pl.BlockSpec((DH, DH), lambda i: (0, 0))],
        out_specs=pl.BlockSpec((HDH, BN), lambda i: (0, i)),
        out_shape=jax.ShapeDtypeStruct((HDH, NP), jnp.int32),
    )(aggT, ypk, dis, b_col, Wn)


def _tc_tail_body(agg_ref, y_ref, dis_ref, b_ref, batch_ref,
                  wh_ref, bh_ref, wo_ref, bo_ref, out_ref,
                  psum_ref, cnt_ref):
    i = pl.program_id(0)
    d = dis_ref[...]
    y = _unpack_pairs(y_ref[...])
    h = jax.nn.gelu((agg_ref[...] + y) * d + b_ref[...])
    bp = batch_ref[...]
    ids = lax.broadcasted_iota(jnp.int32, (G, BN), 0)
    oh = (ids == bp).astype(jnp.float32)
    part = lax.dot_general(h, oh, (((1,), (1,)), ((), ())),
                           preferred_element_type=jnp.float32)
    pcnt = lax.dot_general(jnp.ones((1, BN), jnp.float32), oh,
                           (((1,), (1,)), ((), ())),
                           preferred_element_type=jnp.float32)

    @pl.when(i == 0)
    def _():
        psum_ref[...] = part
        cnt_ref[...] = pcnt

    @pl.when(i != 0)
    def _():
        psum_ref[...] += part
        cnt_ref[...] += pcnt

    @pl.when(i == NP // BN - 1)
    def _():
        pooled = psum_ref[...] / jnp.maximum(cnt_ref[...], 1.0)
        hh = lax.dot_general(pooled, wh_ref[...], (((0,), (0,)), ((), ())),
                             preferred_element_type=jnp.float32)
        hh = jax.nn.gelu(hh + bh_ref[...])
        out = lax.dot_general(hh, wo_ref[...], (((1,), (0,)), ((), ())),
                              preferred_element_type=jnp.float32)
        out_ref[...] = out + bo_ref[...]


def _tc_tail(aggT, ypk, dis, b_col, batch_row, Wh, bh_row, Wo, bo_row):
    return pl.pallas_call(
        _tc_tail_body,
        grid=(NP // BN,),
        in_specs=[pl.BlockSpec((DH, BN), lambda i: (0, i)),
                  pl.BlockSpec((HDH, BN), lambda i: (0, i)),
                  pl.BlockSpec((1, BN), lambda i: (0, i)),
                  pl.BlockSpec((DH, 1), lambda i: (0, 0)),
                  pl.BlockSpec((1, BN), lambda i: (0, i)),
                  pl.BlockSpec((DH, DHL), lambda i: (0, 0)),
                  pl.BlockSpec((1, DHL), lambda i: (0, 0)),
                  pl.BlockSpec((DHL, DOUT), lambda i: (0, 0)),
                  pl.BlockSpec((1, DOUT), lambda i: (0, 0))],
        out_specs=pl.BlockSpec((G, DOUT), lambda i: (0, 0)),
        out_shape=jax.ShapeDtypeStruct((G, DOUT), jnp.float32),
        scratch_shapes=[pltpu.VMEM((DH, G), jnp.float32),
                        pltpu.VMEM((1, G), jnp.float32)],
    )(aggT, ypk, dis, b_col, batch_row, Wh, bh_row, Wo, bo_row)


def kernel(x, edge_index, batch, W1, b1, W2, b2, W3, b3, Wh, bh, Wo, bo):
    src = edge_index[0].astype(jnp.int32)
    dst = edge_index[1].astype(jnp.int32)
    x_p = jnp.pad(x, ((0, NP - N), (0, 0)))
    batch_row = jnp.pad(batch.astype(jnp.int32), (0, NP - N),
                        constant_values=G).reshape(1, NP)

    key = dst & (L - 1)
    perm = jnp.argsort(key, stable=True)
    ksort = key[perm]
    starts = jnp.cumsum(jnp.bincount(ksort, length=L)) - jnp.bincount(ksort, length=L)
    rank = jnp.arange(E, dtype=jnp.int32) - starts[ksort].astype(jnp.int32)
    slot = rank * L + ksort
    perm2 = perm[jnp.argsort(slot)]
    pk_pad = jnp.pad(((dst << SHIFT) | src)[perm2], (0, CHUNK))

    cnt_parts = _sc_degree_fn()(dst)
    ypk1, dis = _tc_first(x_p, W1, cnt_parts)
    _agg = _sc_agg_fn()
    agg1 = _agg(ypk1.reshape(PKROW, ROWW), pk_pad).reshape(DH, NP)
    ypk2 = _tc_mid(agg1, ypk1, dis, b1.reshape(DH, 1), W2)
    agg2 = _agg(ypk2.reshape(PKROW, ROWW), pk_pad).reshape(DH, NP)
    ypk3 = _tc_mid(agg2, ypk2, dis, b2.reshape(DH, 1), W3)
    agg3 = _agg(ypk3.reshape(PKROW, ROWW), pk_pad).reshape(DH, NP)
    return _tc_tail(agg3, ypk3, dis, b3.reshape(DH, 1), batch_row,
                    Wh, bh.reshape(1, DHL), Wo, bo.reshape(1, DOUT))


# speedup vs baseline: 1.1767x; 1.1767x over previous
import functools

import jax
import jax.numpy as jnp
from jax import lax
from jax.experimental import pallas as pl
from jax.experimental.pallas import tpu as pltpu
from jax.experimental.pallas import tpu_sc as plsc

N = 10000
NP = 10240
E = 160000
D_IN = 256
DH = 512
DHL = 1024
DOUT = 128
G = 64

NC, NS, L = 2, 16, 16
NW = NC * NS
HDH = DH // 2
FPP = 4
NPASS = 2
ROWW = FPP * NP
PKROW = HDH // FPP
NROW = DH // FPP
EC = E // NW
CHUNK = 3200
NCHUNK = E // CHUNK
SHIFT = 14
BN = 1024

_sc_params = pltpu.CompilerParams(needs_layout_passes=False)


def _vmesh():
    return plsc.VectorSubcoreMesh(core_axis_name="c", subcore_axis_name="s",
                                  num_cores=NC, num_subcores=NS)


@functools.cache
def _sc_degree_fn():
    return functools.partial(
        pl.kernel,
        out_type=jax.ShapeDtypeStruct((NW, NP), jnp.float32),
        mesh=_vmesh(),
        scratch_types=[pltpu.VMEM((NP,), jnp.float32),
                       pltpu.VMEM((EC + L,), jnp.int32)],
        compiler_params=_sc_params,
    )(_sc_degree_body)


def _sc_degree_body(dst_hbm, out_hbm, cnt_v, dst_v):
    wid = lax.axis_index("s") * NC + lax.axis_index("c")

    def zb(i, c):
        cnt_v[pl.ds(i * L, L)] = jnp.zeros((L,), jnp.float32)
        return c
    lax.fori_loop(0, NP // L, zb, 0)

    tail = (EC // L) * L
    dst_v[pl.ds(tail, L)] = jnp.zeros((L,), jnp.int32)
    pltpu.sync_copy(dst_hbm.at[pl.ds(wid * EC, EC)], dst_v.at[pl.ds(0, EC)])

    ones = jnp.ones((L,), jnp.float32)

    def body(g, c):
        d = dst_v[pl.ds(g * L, L)]
        plsc.addupdate_scatter(cnt_v, [d], ones)
        return c
    lax.fori_loop(0, EC // L, body, 0)
    dtl = dst_v[pl.ds(tail, L)]
    mask = lax.iota(jnp.int32, L) < (EC % L)
    plsc.addupdate_scatter(cnt_v, [dtl], ones, mask=mask)

    pltpu.sync_copy(cnt_v, out_hbm.at[wid])


@functools.cache
def _sc_agg_fn():
    return functools.partial(
        pl.kernel,
        out_type=jax.ShapeDtypeStruct((NROW, ROWW), jnp.float32),
        mesh=_vmesh(),
        scratch_types=[pltpu.VMEM((ROWW,), jnp.int32),
                       pltpu.VMEM((2 * ROWW,), jnp.float32),
                       pltpu.VMEM((CHUNK,), jnp.int32),
                       pltpu.VMEM((CHUNK,), jnp.int32),
                       pltpu.SemaphoreType.DMA,
                       pltpu.SemaphoreType.DMA],
        compiler_params=_sc_params,
    )(_sc_agg_body)


def _sc_agg_body(y_hbm, pk_hbm, out_hbm, y_v, acc_v, eb0, eb1, sem0, sem1):
    wid = lax.axis_index("s") * NC + lax.axis_index("c")

    def process(buf):
        @plsc.parallel_loop(0, CHUNK // L, unroll=4)
        def _(g2):
            e = buf[pl.ds(g2 * L, L)]
            s = e & (2 ** SHIFT - 1)
            d = e >> SHIFT
            for pr in range(FPP):
                g = plsc.load_gather(y_v, [s + (pr * NP)])
                flo = plsc.bitcast(g << 16, jnp.float32)
                fhi = plsc.bitcast(g & jnp.int32(-65536), jnp.float32)
                plsc.addupdate_scatter(acc_v, [d + (pr * NP)], flo)
                plsc.addupdate_scatter(acc_v, [d + ((FPP + pr) * NP)], fhi)

    def pass_body(p, c0):
        row = wid * NPASS + p
        pltpu.make_async_copy(pk_hbm.at[pl.ds(0, CHUNK)], eb0, sem0).start()
        pltpu.sync_copy(y_hbm.at[row], y_v)

        @plsc.parallel_loop(0, 2 * ROWW // L, unroll=8)
        def _(i):
            acc_v[pl.ds(i * L, L)] = jnp.zeros((L,), jnp.float32)

        def chunk_pair(i, c):
            ch0 = i * 2
            pltpu.make_async_copy(
                pk_hbm.at[pl.ds((ch0 + 1) * CHUNK, CHUNK)], eb1, sem1).start()
            pltpu.make_async_copy(
                pk_hbm.at[pl.ds(ch0 * CHUNK, CHUNK)], eb0, sem0).wait()
            process(eb0)
            pltpu.make_async_copy(
                pk_hbm.at[pl.ds((ch0 + 2) * CHUNK, CHUNK)], eb0, sem0).start()
            pltpu.make_async_copy(
                pk_hbm.at[pl.ds((ch0 + 1) * CHUNK, CHUNK)], eb1, sem1).wait()
            process(eb1)
            return c
        lax.fori_loop(0, NCHUNK // 2, chunk_pair, 0)
        pltpu.make_async_copy(
            pk_hbm.at[pl.ds(NCHUNK * CHUNK, CHUNK)], eb0, sem0).wait()

        pltpu.sync_copy(acc_v.at[pl.ds(0, ROWW)], out_hbm.at[row])
        pltpu.sync_copy(acc_v.at[pl.ds(ROWW, ROWW)], out_hbm.at[PKROW + row])
        return c0
    lax.fori_loop(0, NPASS, pass_body, 0)


def _pack_pairs(y):
    v = lax.bitcast_convert_type(y, jnp.uint32)
    r = (v + jnp.uint32(0x7FFF) + ((v >> 16) & jnp.uint32(1))) >> 16
    pk = r[:HDH, :] | (r[HDH:, :] << 16)
    return lax.bitcast_convert_type(pk, jnp.int32)


def _unpack_pairs(pk):
    u = lax.bitcast_convert_type(pk, jnp.uint32)
    lo = lax.bitcast_convert_type(u << 16, jnp.float32)
    hi = lax.bitcast_convert_type(u & jnp.uint32(0xFFFF0000), jnp.float32)
    return jnp.concatenate([lo, hi], axis=0)


def _tc_first_body(x_ref, w_ref, cnt_ref, y_ref, dis_ref):
    c = jnp.sum(cnt_ref[...], axis=0, keepdims=True)
    d = lax.rsqrt(c + 1.0)
    y = lax.dot_general(w_ref[...], x_ref[...], (((0,), (1,)), ((), ())),
                        preferred_element_type=jnp.float32)
    y_ref[...] = _pack_pairs(y * d)
    dis_ref[...] = d


def _tc_first(x_p, W1, cnt_parts):
    return pl.pallas_call(
        _tc_first_body,
        grid=(NP // BN,),
        in_specs=[pl.BlockSpec((BN, D_IN), lambda i: (i, 0)),
                  pl.BlockSpec((D_IN, DH), lambda i: (0, 0)),
                  pl.BlockSpec((NW, BN), lambda i: (0, i))],
        out_specs=[pl.BlockSpec((HDH, BN), lambda i: (0, i)),
                   pl.BlockSpec((1, BN), lambda i: (0, i))],
        out_shape=[jax.ShapeDtypeStruct((HDH, NP), jnp.int32),
                   jax.ShapeDtypeStruct((1, NP), jnp.float32)],
    )(x_p, W1, cnt_parts)


def _tc_mid_body(agg_ref, y_ref, dis_ref, b_ref, w_ref, out_ref):
    d = dis_ref[...]
    y = _unpack_pairs(y_ref[...])
    h = jax.nn.gelu((agg_ref[...] + y) * d + b_ref[...])
    yn = lax.dot_general(w_ref[...], h, (((0,), (0,)), ((), ())),
                         preferred_element_type=jnp.float32)
    out_ref[...] = _pack_pairs(yn * d)


def _tc_mid(aggT, ypk, dis, b_col, Wn):
    return pl.pallas_call(
        _tc_mid_body,
        grid=(NP // BN,),
        in_specs=[pl.BlockSpec((DH, BN), lambda i: (0, i)),
                  pl.BlockSpec((HDH, BN), lambda i: (0, i)),
                  pl.BlockSpec((1, BN), lambda i: (0, i)),
                  pl.BlockSpec((DH, 1), lambda i: (0, 0)),
                  pl.BlockSpec((DH, DH), lambda i: (0, 0))],
        out_specs=pl.BlockSpec((HDH, BN), lambda i: (0, i)),
        out_shape=jax.ShapeDtypeStruct((HDH, NP), jnp.int32),
    )(aggT, ypk, dis, b_col, Wn)


def _tc_tail_body(agg_ref, y_ref, dis_ref, b_ref, batch_ref,
                  wh_ref, bh_ref, wo_ref, bo_ref, out_ref,
                  psum_ref, cnt_ref):
    i = pl.program_id(0)
    d = dis_ref[...]
    y = _unpack_pairs(y_ref[...])
    h = jax.nn.gelu((agg_ref[...] + y) * d + b_ref[...])
    bp = batch_ref[...]
    ids = lax.broadcasted_iota(jnp.int32, (G, BN), 0)
    oh = (ids == bp).astype(jnp.float32)
    part = lax.dot_general(h, oh, (((1,), (1,)), ((), ())),
                           preferred_element_type=jnp.float32)
    pcnt = lax.dot_general(jnp.ones((1, BN), jnp.float32), oh,
                           (((1,), (1,)), ((), ())),
                           preferred_element_type=jnp.float32)

    @pl.when(i == 0)
    def _():
        psum_ref[...] = part
        cnt_ref[...] = pcnt

    @pl.when(i != 0)
    def _():
        psum_ref[...] += part
        cnt_ref[...] += pcnt

    @pl.when(i == NP // BN - 1)
    def _():
        pooled = psum_ref[...] / jnp.maximum(cnt_ref[...], 1.0)
        hh = lax.dot_general(pooled, wh_ref[...], (((0,), (0,)), ((), ())),
                             preferred_element_type=jnp.float32)
        hh = jax.nn.gelu(hh + bh_ref[...])
        out = lax.dot_general(hh, wo_ref[...], (((1,), (0,)), ((), ())),
                              preferred_element_type=jnp.float32)
        out_ref[...] = out + bo_ref[...]


def _tc_tail(aggT, ypk, dis, b_col, batch_row, Wh, bh_row, Wo, bo_row):
    return pl.pallas_call(
        _tc_tail_body,
        grid=(NP // BN,),
        in_specs=[pl.BlockSpec((DH, BN), lambda i: (0, i)),
                  pl.BlockSpec((HDH, BN), lambda i: (0, i)),
                  pl.BlockSpec((1, BN), lambda i: (0, i)),
                  pl.BlockSpec((DH, 1), lambda i: (0, 0)),
                  pl.BlockSpec((1, BN), lambda i: (0, i)),
                  pl.BlockSpec((DH, DHL), lambda i: (0, 0)),
                  pl.BlockSpec((1, DHL), lambda i: (0, 0)),
                  pl.BlockSpec((DHL, DOUT), lambda i: (0, 0)),
                  pl.BlockSpec((1, DOUT), lambda i: (0, 0))],
        out_specs=pl.BlockSpec((G, DOUT), lambda i: (0, 0)),
        out_shape=jax.ShapeDtypeStruct((G, DOUT), jnp.float32),
        scratch_shapes=[pltpu.VMEM((DH, G), jnp.float32),
                        pltpu.VMEM((1, G), jnp.float32)],
    )(aggT, ypk, dis, b_col, batch_row, Wh, bh_row, Wo, bo_row)


def kernel(x, edge_index, batch, W1, b1, W2, b2, W3, b3, Wh, bh, Wo, bo):
    src = edge_index[0].astype(jnp.int32)
    dst = edge_index[1].astype(jnp.int32)
    x_p = jnp.pad(x, ((0, NP - N), (0, 0)))
    batch_row = jnp.pad(batch.astype(jnp.int32), (0, NP - N),
                        constant_values=G).reshape(1, NP)

    pk_pad = jnp.pad((dst << SHIFT) | src, (0, CHUNK))

    cnt_parts = _sc_degree_fn()(dst)
    ypk1, dis = _tc_first(x_p, W1, cnt_parts)
    _agg = _sc_agg_fn()
    agg1 = _agg(ypk1.reshape(PKROW, ROWW), pk_pad).reshape(DH, NP)
    ypk2 = _tc_mid(agg1, ypk1, dis, b1.reshape(DH, 1), W2)
    agg2 = _agg(ypk2.reshape(PKROW, ROWW), pk_pad).reshape(DH, NP)
    ypk3 = _tc_mid(agg2, ypk2, dis, b2.reshape(DH, 1), W3)
    agg3 = _agg(ypk3.reshape(PKROW, ROWW), pk_pad).reshape(DH, NP)
    return _tc_tail(agg3, ypk3, dis, b3.reshape(DH, 1), batch_row,
                    Wh, bh.reshape(1, DHL), Wo, bo.reshape(1, DOUT))
